# baseline (speedup 1.0000x reference)
import functools

import jax
import jax.numpy as jnp
from jax import lax
from jax.experimental import pallas as pl
from jax.experimental.pallas import tpu as pltpu
from jax.experimental.pallas import tpu_sc as plsc


def _pack_halves(y):
    bits = lax.bitcast_convert_type(y, jnp.int32)
    H = y.shape[1] // 2
    return lax.shift_right_logical(bits[:, :H], 16) | bits[:, H:]


def _unpack_halves(p):
    a = lax.bitcast_convert_type(p << 16, jnp.float32)
    b = lax.bitcast_convert_type(p & jnp.int32(-65536), jnp.float32)
    return a, b


def _sc_gather(table, idx, chunk=32, nbuf=4):
    B = idx.shape[0]
    D = table.shape[1]
    info = plsc.get_sparse_core_info()
    NC, NS = info.num_cores, info.num_subcores
    NW = NC * NS
    bpw = B // NW
    nch = bpw // chunk
    nbuf = min(nbuf, nch)
    idx3 = idx.reshape(NW, nch, chunk)

    scratch = [pltpu.VMEM((nch, chunk), jnp.int32)]
    scratch += [pltpu.VMEM((chunk, D), table.dtype) for _ in range(nbuf)]
    scratch += [pltpu.SemaphoreType.DMA for _ in range(2 * nbuf)]

    @functools.partial(
        pl.kernel,
        mesh=plsc.VectorSubcoreMesh(core_axis_name="c", subcore_axis_name="s"),
        out_type=jax.ShapeDtypeStruct((B, D), table.dtype),
        scratch_types=scratch,
    )
    def k(table_h, idx_h, out_h, idx_v, *rest):
        bufs = rest[:nbuf]
        gsem = rest[nbuf:2 * nbuf]
        osem = rest[2 * nbuf:3 * nbuf]
        wid = lax.axis_index("s") * NC + lax.axis_index("c")
        base = wid * bpw
        pltpu.sync_copy(idx_h.at[wid], idx_v)
        in_fl = {}
        out_fl = {}
        for b in range(nbuf):
            in_fl[b] = pltpu.async_copy(table_h.at[idx_v.at[b]], bufs[b], gsem[b])
        for i in range(nch):
            b = i % nbuf
            in_fl[b].wait()
            out_fl[b] = pltpu.async_copy(
                bufs[b], out_h.at[pl.ds(base + i * chunk, chunk)], osem[b])
            if i + nbuf < nch:
                out_fl[b].wait()
                in_fl[b] = pltpu.async_copy(
                    table_h.at[idx_v.at[i + nbuf]], bufs[b], gsem[b])
        for b in range(nbuf):
            out_fl[b].wait()

    return k(table, idx3)


def _group_onehot(NG, K, G):
    gidx = lax.broadcasted_iota(jnp.int32, (NG, K), 0)
    lidx = lax.broadcasted_iota(jnp.int32, (NG, K), 1)
    return (lidx // G == gidx).astype(jnp.float32)


def _deq_pack_body(x_ref, is_ref, o_ref, *, G):
    B, K = x_ref.shape
    NG = K // G
    srep = jnp.dot(is_ref[...], _group_onehot(NG, K, G),
                   preferred_element_type=jnp.float32)
    xq = (x_ref[...].astype(jnp.float32) * srep)
    o_ref[...] = _pack_halves(xq.astype(jnp.bfloat16).astype(jnp.float32))


def _dequant_pack(x, is_, G):
    T, K = x.shape
    NG = K // G
    B = 512
    return pl.pallas_call(
        functools.partial(_deq_pack_body, G=G),
        grid=(T // B,),
        in_specs=[
            pl.BlockSpec((B, K), lambda i: (i, 0)),
            pl.BlockSpec((B, NG), lambda i: (i, 0)),
        ],
        out_specs=pl.BlockSpec((B, K // 2), lambda i: (i, 0)),
        out_shape=jax.ShapeDtypeStruct((T, K // 2), jnp.int32),
        compiler_params=pltpu.CompilerParams(
            dimension_semantics=("arbitrary",)),
    )(x, is_)


def _gemm_body(xp_ref, w_ref, ws_ref, b_ref, o_ref, *, G):
    C, H = xp_ref.shape
    K = 2 * H
    NG = K // G
    NB = w_ref.shape[2]
    xa, xb = _unpack_halves(xp_ref[...])
    x = jnp.concatenate([xa, xb], axis=1).astype(jnp.bfloat16)
    ws = ws_ref[0]
    wsrep = jnp.broadcast_to(ws[:, None, :], (NG, G, NB)).reshape(K, NB)
    wq = (w_ref[0].astype(jnp.float32) * wsrep).astype(jnp.bfloat16)
    acc = jnp.dot(x, wq, preferred_element_type=jnp.float32)
    y = (acc + b_ref[0]).astype(jnp.bfloat16).astype(jnp.float32)
    o_ref[...] = _pack_halves(y)


def _gemm_body_alias(yin_ref, *refs, G):
    _gemm_body(*refs, G=G)


def _gemm_part(y_prev, xp, weight, weight_scale, bias, C, G, e_off):
    E, K, N = weight.shape
    NG = K // G
    ne = xp.shape[0] // C
    NS = 2
    NB = N // NS
    specs = [
        pl.BlockSpec((C, K // 2), lambda e, n: (e, 0)),
        pl.BlockSpec((1, K, NB), lambda e, n: (e + e_off, 0, n)),
        pl.BlockSpec((1, NG, NB), lambda e, n: (e + e_off, 0, n)),
        pl.BlockSpec((1, 1, NB), lambda e, n: (e + e_off, 0, n)),
    ]
    args = [xp, weight, weight_scale, bias.reshape(E, 1, N)]
    io_alias = {}
    body = functools.partial(_gemm_body, G=G)
    if y_prev is not None:
        specs = [pl.BlockSpec(memory_space=pltpu.MemorySpace.HBM)] + specs
        args = [y_prev] + args
        io_alias = {0: 0}
        body = functools.partial(_gemm_body_alias, G=G)
    return pl.pallas_call(
        body,
        grid=(ne, NS),
        in_specs=specs,
        out_specs=pl.BlockSpec((C, NB // 2), lambda e, n: (e + e_off, n)),
        out_shape=jax.ShapeDtypeStruct((E * C, N // 2), jnp.int32),
        input_output_aliases=io_alias,
        compiler_params=pltpu.CompilerParams(
            dimension_semantics=("arbitrary", "arbitrary")),
    )(*args)


def _combine_body(y_ref, g_ref, sh_ref, rsf_ref, o_ref):
    N = o_ref.shape[1]
    H = N // 2
    g = g_ref[...]
    rsf = rsf_ref[0, 0]
    s = jnp.maximum(g[:, 0:1] + g[:, 1:2], 1e-12)
    w = g * (rsf / s)
    p = y_ref[...]
    Q = N // 4

    def quarters(ph):
        a0, a1 = _unpack_halves(ph[:, :Q])
        b0, b1 = _unpack_halves(ph[:, Q:])
        return (a0, a1, b0, b1)

    y0 = quarters(p[:, :H])
    y1 = quarters(p[:, H:])
    w0, w1 = w[:, 0:1], w[:, 1:2]
    sh = sh_ref[...]
    parts = [w0 * y0[q] + w1 * y1[q] + sh[:, Q * q:Q * (q + 1)]
             for q in range(4)]
    o_ref[...] = jnp.concatenate(parts, axis=1).astype(jnp.bfloat16)


def _combine(yp2, gates, shared, rsf):
    T, N = yp2.shape
    B = 512
    return pl.pallas_call(
        _combine_body,
        grid=(T // B,),
        in_specs=[
            pl.BlockSpec((B, N), lambda i: (i, 0)),
            pl.BlockSpec((B, 2), lambda i: (i, 0)),
            pl.BlockSpec((B, N), lambda i: (i, 0)),
            pl.BlockSpec((1, 1), lambda i: (0, 0)),
        ],
        out_specs=pl.BlockSpec((B, N), lambda i: (i, 0)),
        out_shape=jax.ShapeDtypeStruct((T, N), jnp.bfloat16),
        compiler_params=pltpu.CompilerParams(
            dimension_semantics=("arbitrary",)),
    )(yp2, gates, shared, rsf)


def kernel(input, weight, top_k_gates, token_indices, src_to_dst, token_count,
           shared_output, weight_scale, input_scale, bias,
           routed_scaling_factor):
    T, K = input.shape
    E, _, N = weight.shape
    total = token_indices.shape[0]
    C = total // E
    G = K // input_scale.shape[1]
    xp = _dequant_pack(input, input_scale, G)
    xdp = _sc_gather(xp, token_indices)
    yp = _gemm_part(None, xdp, weight, weight_scale, bias, C, G, 0)
    ysp = _sc_gather(yp, src_to_dst.reshape(-1))
    rsf = jnp.asarray(routed_scaling_factor, jnp.float32).reshape(1, 1)
    return _combine(ysp.reshape(T, N), top_k_gates, shared_output, rsf)


# speedup vs baseline: 1.1888x; 1.1888x over previous
import functools

import jax
import jax.numpy as jnp
from jax import lax
from jax.experimental import pallas as pl
from jax.experimental.pallas import tpu as pltpu
from jax.experimental.pallas import tpu_sc as plsc


def _pack_halves(y):
    bits = lax.bitcast_convert_type(y, jnp.int32)
    H = y.shape[1] // 2
    return lax.shift_right_logical(bits[:, :H], 16) | bits[:, H:]


def _unpack_halves(p):
    a = lax.bitcast_convert_type(p << 16, jnp.float32)
    b = lax.bitcast_convert_type(p & jnp.int32(-65536), jnp.float32)
    return a, b


def _sc_gather(table, idx, chunk=32, nbuf=4):
    B = idx.shape[0]
    D = table.shape[1]
    info = plsc.get_sparse_core_info()
    NC, NS = info.num_cores, info.num_subcores
    NW = NC * NS
    bpw = B // NW
    nch = bpw // chunk
    nbuf = min(nbuf, nch)
    idx3 = idx.reshape(NW, nch, chunk)

    scratch = [pltpu.VMEM((nch, chunk), jnp.int32)]
    scratch += [pltpu.VMEM((chunk, D), table.dtype) for _ in range(nbuf)]
    scratch += [pltpu.SemaphoreType.DMA for _ in range(2 * nbuf)]

    @functools.partial(
        pl.kernel,
        mesh=plsc.VectorSubcoreMesh(core_axis_name="c", subcore_axis_name="s"),
        out_type=jax.ShapeDtypeStruct((B, D), table.dtype),
        scratch_types=scratch,
    )
    def k(table_h, idx_h, out_h, idx_v, *rest):
        bufs = rest[:nbuf]
        gsem = rest[nbuf:2 * nbuf]
        osem = rest[2 * nbuf:3 * nbuf]
        wid = lax.axis_index("s") * NC + lax.axis_index("c")
        base = wid * bpw
        pltpu.sync_copy(idx_h.at[wid], idx_v)
        in_fl = {}
        out_fl = {}
        for b in range(nbuf):
            in_fl[b] = pltpu.async_copy(table_h.at[idx_v.at[b]], bufs[b], gsem[b])
        for i in range(nch):
            b = i % nbuf
            in_fl[b].wait()
            out_fl[b] = pltpu.async_copy(
                bufs[b], out_h.at[pl.ds(base + i * chunk, chunk)], osem[b])
            if i + nbuf < nch:
                out_fl[b].wait()
                in_fl[b] = pltpu.async_copy(
                    table_h.at[idx_v.at[i + nbuf]], bufs[b], gsem[b])
        for b in range(nbuf):
            out_fl[b].wait()

    return k(table, idx3)


def _group_onehot(NG, K, G):
    gidx = lax.broadcasted_iota(jnp.int32, (NG, K), 0)
    lidx = lax.broadcasted_iota(jnp.int32, (NG, K), 1)
    return (lidx // G == gidx).astype(jnp.float32)


def _deq_pack_body(x_ref, is_ref, o_ref, *, G):
    B, K = x_ref.shape
    NG = K // G
    srep = jnp.dot(is_ref[...], _group_onehot(NG, K, G),
                   preferred_element_type=jnp.float32)
    xq = (x_ref[...].astype(jnp.float32) * srep)
    o_ref[...] = _pack_halves(xq.astype(jnp.bfloat16).astype(jnp.float32))


def _dequant_pack(x, is_, G):
    T, K = x.shape
    NG = K // G
    B = 512
    return pl.pallas_call(
        functools.partial(_deq_pack_body, G=G),
        grid=(T // B,),
        in_specs=[
            pl.BlockSpec((B, K), lambda i: (i, 0)),
            pl.BlockSpec((B, NG), lambda i: (i, 0)),
        ],
        out_specs=pl.BlockSpec((B, K // 2), lambda i: (i, 0)),
        out_shape=jax.ShapeDtypeStruct((T, K // 2), jnp.int32),
        compiler_params=pltpu.CompilerParams(
            dimension_semantics=("arbitrary",)),
    )(x, is_)


def _gemm_body(xp_ref, w_ref, ws_ref, b_ref, o_ref, *, G):
    C, H = xp_ref.shape
    K = 2 * H
    NG = K // G
    NB = w_ref.shape[2]
    xa, xb = _unpack_halves(xp_ref[...])
    x = jnp.concatenate([xa, xb], axis=1).astype(jnp.bfloat16)
    ws = ws_ref[0]
    wsrep = jnp.broadcast_to(ws[:, None, :], (NG, G, NB)).reshape(K, NB)
    wq = (w_ref[0].astype(jnp.float32) * wsrep).astype(jnp.bfloat16)
    acc = jnp.dot(x, wq, preferred_element_type=jnp.float32)
    y = (acc + b_ref[0]).astype(jnp.bfloat16).astype(jnp.float32)
    o_ref[...] = _pack_halves(y)


def _gemm_body_alias(yin_ref, *refs, G):
    _gemm_body(*refs, G=G)


def _gemm_part(y_prev, xp, weight, weight_scale, bias, C, G, e_off):
    E, K, N = weight.shape
    NG = K // G
    ne = xp.shape[0] // C
    NS = 1
    NB = N // NS
    specs = [
        pl.BlockSpec((C, K // 2), lambda e, n: (e, 0)),
        pl.BlockSpec((1, K, NB), lambda e, n: (e + e_off, 0, n)),
        pl.BlockSpec((1, NG, NB), lambda e, n: (e + e_off, 0, n)),
        pl.BlockSpec((1, 1, NB), lambda e, n: (e + e_off, 0, n)),
    ]
    args = [xp, weight, weight_scale, bias.reshape(E, 1, N)]
    io_alias = {}
    body = functools.partial(_gemm_body, G=G)
    if y_prev is not None:
        specs = [pl.BlockSpec(memory_space=pltpu.MemorySpace.HBM)] + specs
        args = [y_prev] + args
        io_alias = {0: 0}
        body = functools.partial(_gemm_body_alias, G=G)
    return pl.pallas_call(
        body,
        grid=(ne, NS),
        in_specs=specs,
        out_specs=pl.BlockSpec((C, NB // 2), lambda e, n: (e + e_off, n)),
        out_shape=jax.ShapeDtypeStruct((E * C, N // 2), jnp.int32),
        input_output_aliases=io_alias,
        compiler_params=pltpu.CompilerParams(
            dimension_semantics=("arbitrary", "arbitrary")),
    )(*args)


def _combine_body(y_ref, g_ref, sh_ref, rsf_ref, o_ref):
    N = o_ref.shape[1]
    H = N // 2
    g = g_ref[...]
    rsf = rsf_ref[0, 0]
    s = jnp.maximum(g[:, 0:1] + g[:, 1:2], 1e-12)
    w = g * (rsf / s)
    p = y_ref[...]
    y0a, y0b = _unpack_halves(p[:, :H])
    y1a, y1b = _unpack_halves(p[:, H:])
    w0, w1 = w[:, 0:1], w[:, 1:2]
    sh = sh_ref[...]
    oa = w0 * y0a + w1 * y1a + sh[:, :H]
    ob = w0 * y0b + w1 * y1b + sh[:, H:]
    o_ref[...] = jnp.concatenate([oa, ob], axis=1).astype(jnp.bfloat16)


def _combine(yp2, gates, shared, rsf):
    T, N = yp2.shape
    B = 512
    return pl.pallas_call(
        _combine_body,
        grid=(T // B,),
        in_specs=[
            pl.BlockSpec((B, N), lambda i: (i, 0)),
            pl.BlockSpec((B, 2), lambda i: (i, 0)),
            pl.BlockSpec((B, N), lambda i: (i, 0)),
            pl.BlockSpec((1, 1), lambda i: (0, 0)),
        ],
        out_specs=pl.BlockSpec((B, N), lambda i: (i, 0)),
        out_shape=jax.ShapeDtypeStruct((T, N), jnp.bfloat16),
        compiler_params=pltpu.CompilerParams(
            dimension_semantics=("arbitrary",)),
    )(yp2, gates, shared, rsf)


def kernel(input, weight, top_k_gates, token_indices, src_to_dst, token_count,
           shared_output, weight_scale, input_scale, bias,
           routed_scaling_factor):
    T, K = input.shape
    E, _, N = weight.shape
    total = token_indices.shape[0]
    C = total // E
    G = K // input_scale.shape[1]
    xp = _dequant_pack(input, input_scale, G)
    xdp = _sc_gather(xp, token_indices)
    yp = _gemm_part(None, xdp, weight, weight_scale, bias, C, G, 0)
    ysp = _sc_gather(yp, src_to_dst.reshape(-1))
    rsf = jnp.asarray(routed_scaling_factor, jnp.float32).reshape(1, 1)
    return _combine(ysp.reshape(T, N), top_k_gates, shared_output, rsf)


# speedup vs baseline: 1.2026x; 1.0116x over previous
import functools

import jax
import jax.numpy as jnp
from jax import lax
from jax.experimental import pallas as pl
from jax.experimental.pallas import tpu as pltpu
from jax.experimental.pallas import tpu_sc as plsc


def _pack_halves(y):
    bits = lax.bitcast_convert_type(y, jnp.int32)
    H = y.shape[1] // 2
    return lax.shift_right_logical(bits[:, :H], 16) | bits[:, H:]


def _unpack_halves(p):
    a = lax.bitcast_convert_type(p << 16, jnp.float32)
    b = lax.bitcast_convert_type(p & jnp.int32(-65536), jnp.float32)
    return a, b


def _sc_gather(table, idx, chunk=32, nbuf=4):
    B = idx.shape[0]
    D = table.shape[1]
    info = plsc.get_sparse_core_info()
    NC, NS = info.num_cores, info.num_subcores
    NW = NC * NS
    bpw = B // NW
    nch = bpw // chunk
    nbuf = min(nbuf, nch)
    idx3 = idx.reshape(NW, nch, chunk)

    scratch = [pltpu.VMEM((nch, chunk), jnp.int32)]
    scratch += [pltpu.VMEM((chunk, D), table.dtype) for _ in range(nbuf)]
    scratch += [pltpu.SemaphoreType.DMA for _ in range(2 * nbuf)]

    @functools.partial(
        pl.kernel,
        mesh=plsc.VectorSubcoreMesh(core_axis_name="c", subcore_axis_name="s"),
        out_type=jax.ShapeDtypeStruct((B, D), table.dtype),
        scratch_types=scratch,
    )
    def k(table_h, idx_h, out_h, idx_v, *rest):
        bufs = rest[:nbuf]
        gsem = rest[nbuf:2 * nbuf]
        osem = rest[2 * nbuf:3 * nbuf]
        wid = lax.axis_index("s") * NC + lax.axis_index("c")
        base = wid * bpw
        pltpu.sync_copy(idx_h.at[wid], idx_v)
        in_fl = {}
        out_fl = {}
        for b in range(nbuf):
            in_fl[b] = pltpu.async_copy(table_h.at[idx_v.at[b]], bufs[b], gsem[b])
        for i in range(nch):
            b = i % nbuf
            in_fl[b].wait()
            out_fl[b] = pltpu.async_copy(
                bufs[b], out_h.at[pl.ds(base + i * chunk, chunk)], osem[b])
            if i + nbuf < nch:
                out_fl[b].wait()
                in_fl[b] = pltpu.async_copy(
                    table_h.at[idx_v.at[i + nbuf]], bufs[b], gsem[b])
        for b in range(nbuf):
            out_fl[b].wait()

    return k(table, idx3)


def _group_onehot(NG, K, G):
    gidx = lax.broadcasted_iota(jnp.int32, (NG, K), 0)
    lidx = lax.broadcasted_iota(jnp.int32, (NG, K), 1)
    return (lidx // G == gidx).astype(jnp.float32)


def _deq_pack_body(x_ref, is_ref, o_ref, *, G):
    B, K = x_ref.shape
    NG = K // G
    srep = jnp.dot(is_ref[...], _group_onehot(NG, K, G),
                   preferred_element_type=jnp.float32)
    xq = (x_ref[...].astype(jnp.float32) * srep)
    o_ref[...] = _pack_halves(xq.astype(jnp.bfloat16).astype(jnp.float32))


def _dequant_pack(x, is_, G):
    T, K = x.shape
    NG = K // G
    B = 1024
    return pl.pallas_call(
        functools.partial(_deq_pack_body, G=G),
        grid=(T // B,),
        in_specs=[
            pl.BlockSpec((B, K), lambda i: (i, 0)),
            pl.BlockSpec((B, NG), lambda i: (i, 0)),
        ],
        out_specs=pl.BlockSpec((B, K // 2), lambda i: (i, 0)),
        out_shape=jax.ShapeDtypeStruct((T, K // 2), jnp.int32),
        compiler_params=pltpu.CompilerParams(
            dimension_semantics=("arbitrary",)),
    )(x, is_)


def _gemm_body(xp_ref, w_ref, ws_ref, b_ref, o_ref, *, G):
    C, H = xp_ref.shape
    K = 2 * H
    NG = K // G
    NB = w_ref.shape[2]
    xa, xb = _unpack_halves(xp_ref[...])
    x = jnp.concatenate([xa, xb], axis=1).astype(jnp.bfloat16)
    ws = ws_ref[0]
    wsrep = jnp.broadcast_to(ws[:, None, :], (NG, G, NB)).reshape(K, NB)
    wq = (w_ref[0].astype(jnp.float32) * wsrep).astype(jnp.bfloat16)
    acc = jnp.dot(x, wq, preferred_element_type=jnp.float32)
    y = (acc + b_ref[0]).astype(jnp.bfloat16).astype(jnp.float32)
    o_ref[...] = _pack_halves(y)


def _gemm_body_alias(yin_ref, *refs, G):
    _gemm_body(*refs, G=G)


def _gemm_part(y_prev, xp, weight, weight_scale, bias, C, G, e_off):
    E, K, N = weight.shape
    NG = K // G
    ne = xp.shape[0] // C
    NS = 1
    NB = N // NS
    specs = [
        pl.BlockSpec((C, K // 2), lambda e, n: (e, 0)),
        pl.BlockSpec((1, K, NB), lambda e, n: (e + e_off, 0, n)),
        pl.BlockSpec((1, NG, NB), lambda e, n: (e + e_off, 0, n)),
        pl.BlockSpec((1, 1, NB), lambda e, n: (e + e_off, 0, n)),
    ]
    args = [xp, weight, weight_scale, bias.reshape(E, 1, N)]
    io_alias = {}
    body = functools.partial(_gemm_body, G=G)
    if y_prev is not None:
        specs = [pl.BlockSpec(memory_space=pltpu.MemorySpace.HBM)] + specs
        args = [y_prev] + args
        io_alias = {0: 0}
        body = functools.partial(_gemm_body_alias, G=G)
    return pl.pallas_call(
        body,
        grid=(ne, NS),
        in_specs=specs,
        out_specs=pl.BlockSpec((C, NB // 2), lambda e, n: (e + e_off, n)),
        out_shape=jax.ShapeDtypeStruct((E * C, N // 2), jnp.int32),
        input_output_aliases=io_alias,
        compiler_params=pltpu.CompilerParams(
            dimension_semantics=("arbitrary", "arbitrary")),
    )(*args)


def _combine_body(y_ref, g_ref, sh_ref, rsf_ref, o_ref):
    N = o_ref.shape[1]
    H = N // 2
    g = g_ref[...]
    rsf = rsf_ref[0, 0]
    s = jnp.maximum(g[:, 0:1] + g[:, 1:2], 1e-12)
    w = g * (rsf / s)
    p = y_ref[...]
    y0a, y0b = _unpack_halves(p[:, :H])
    y1a, y1b = _unpack_halves(p[:, H:])
    w0, w1 = w[:, 0:1], w[:, 1:2]
    sh = sh_ref[...]
    oa = w0 * y0a + w1 * y1a + sh[:, :H]
    ob = w0 * y0b + w1 * y1b + sh[:, H:]
    o_ref[...] = jnp.concatenate([oa, ob], axis=1).astype(jnp.bfloat16)


def _combine(yp2, gates, shared, rsf):
    T, N = yp2.shape
    B = 1024
    return pl.pallas_call(
        _combine_body,
        grid=(T // B,),
        in_specs=[
            pl.BlockSpec((B, N), lambda i: (i, 0)),
            pl.BlockSpec((B, 2), lambda i: (i, 0)),
            pl.BlockSpec((B, N), lambda i: (i, 0)),
            pl.BlockSpec((1, 1), lambda i: (0, 0)),
        ],
        out_specs=pl.BlockSpec((B, N), lambda i: (i, 0)),
        out_shape=jax.ShapeDtypeStruct((T, N), jnp.bfloat16),
        compiler_params=pltpu.CompilerParams(
            dimension_semantics=("arbitrary",)),
    )(yp2, gates, shared, rsf)


def kernel(input, weight, top_k_gates, token_indices, src_to_dst, token_count,
           shared_output, weight_scale, input_scale, bias,
           routed_scaling_factor):
    T, K = input.shape
    E, _, N = weight.shape
    total = token_indices.shape[0]
    C = total // E
    G = K // input_scale.shape[1]
    xp = _dequant_pack(input, input_scale, G)
    xdp = _sc_gather(xp, token_indices)
    yp = _gemm_part(None, xdp, weight, weight_scale, bias, C, G, 0)
    ysp = _sc_gather(yp, src_to_dst.reshape(-1))
    rsf = jnp.asarray(routed_scaling_factor, jnp.float32).reshape(1, 1)
    return _combine(ysp.reshape(T, N), top_k_gates, shared_output, rsf)


# speedup vs baseline: 1.2066x; 1.0033x over previous
import functools

import jax
import jax.numpy as jnp
from jax import lax
from jax.experimental import pallas as pl
from jax.experimental.pallas import tpu as pltpu
from jax.experimental.pallas import tpu_sc as plsc


def _pack_halves(y):
    bits = lax.bitcast_convert_type(y, jnp.int32)
    H = y.shape[1] // 2
    return lax.shift_right_logical(bits[:, :H], 16) | bits[:, H:]


def _unpack_halves(p):
    a = lax.bitcast_convert_type(p << 16, jnp.float32)
    b = lax.bitcast_convert_type(p & jnp.int32(-65536), jnp.float32)
    return a, b


def _sc_gather(table, idx, chunk=64, nbuf=3):
    B = idx.shape[0]
    D = table.shape[1]
    info = plsc.get_sparse_core_info()
    NC, NS = info.num_cores, info.num_subcores
    NW = NC * NS
    bpw = B // NW
    nch = bpw // chunk
    nbuf = min(nbuf, nch)
    idx3 = idx.reshape(NW, nch, chunk)

    scratch = [pltpu.VMEM((nch, chunk), jnp.int32)]
    scratch += [pltpu.VMEM((chunk, D), table.dtype) for _ in range(nbuf)]
    scratch += [pltpu.SemaphoreType.DMA for _ in range(2 * nbuf)]

    @functools.partial(
        pl.kernel,
        mesh=plsc.VectorSubcoreMesh(core_axis_name="c", subcore_axis_name="s"),
        out_type=jax.ShapeDtypeStruct((B, D), table.dtype),
        scratch_types=scratch,
    )
    def k(table_h, idx_h, out_h, idx_v, *rest):
        bufs = rest[:nbuf]
        gsem = rest[nbuf:2 * nbuf]
        osem = rest[2 * nbuf:3 * nbuf]
        wid = lax.axis_index("s") * NC + lax.axis_index("c")
        base = wid * bpw
        pltpu.sync_copy(idx_h.at[wid], idx_v)
        in_fl = {}
        out_fl = {}
        for b in range(nbuf):
            in_fl[b] = pltpu.async_copy(table_h.at[idx_v.at[b]], bufs[b], gsem[b])
        for i in range(nch):
            b = i % nbuf
            in_fl[b].wait()
            out_fl[b] = pltpu.async_copy(
                bufs[b], out_h.at[pl.ds(base + i * chunk, chunk)], osem[b])
            if i + nbuf < nch:
                out_fl[b].wait()
                in_fl[b] = pltpu.async_copy(
                    table_h.at[idx_v.at[i + nbuf]], bufs[b], gsem[b])
        for b in range(nbuf):
            out_fl[b].wait()

    return k(table, idx3)


def _group_onehot(NG, K, G):
    gidx = lax.broadcasted_iota(jnp.int32, (NG, K), 0)
    lidx = lax.broadcasted_iota(jnp.int32, (NG, K), 1)
    return (lidx // G == gidx).astype(jnp.float32)


def _deq_pack_body(x_ref, is_ref, o_ref, *, G):
    B, K = x_ref.shape
    NG = K // G
    srep = jnp.dot(is_ref[...], _group_onehot(NG, K, G),
                   preferred_element_type=jnp.float32)
    xq = (x_ref[...].astype(jnp.float32) * srep)
    o_ref[...] = _pack_halves(xq.astype(jnp.bfloat16).astype(jnp.float32))


def _dequant_pack(x, is_, G):
    T, K = x.shape
    NG = K // G
    B = 1024
    return pl.pallas_call(
        functools.partial(_deq_pack_body, G=G),
        grid=(T // B,),
        in_specs=[
            pl.BlockSpec((B, K), lambda i: (i, 0)),
            pl.BlockSpec((B, NG), lambda i: (i, 0)),
        ],
        out_specs=pl.BlockSpec((B, K // 2), lambda i: (i, 0)),
        out_shape=jax.ShapeDtypeStruct((T, K // 2), jnp.int32),
        compiler_params=pltpu.CompilerParams(
            dimension_semantics=("arbitrary",)),
    )(x, is_)


def _gemm_body(xp_ref, w_ref, ws_ref, b_ref, o_ref, *, G):
    C, H = xp_ref.shape
    K = 2 * H
    NG = K // G
    NB = w_ref.shape[2]
    xa, xb = _unpack_halves(xp_ref[...])
    x = jnp.concatenate([xa, xb], axis=1).astype(jnp.bfloat16)
    ws = ws_ref[0]
    wsrep = jnp.broadcast_to(ws[:, None, :], (NG, G, NB)).reshape(K, NB)
    wq = (w_ref[0].astype(jnp.float32) * wsrep).astype(jnp.bfloat16)
    acc = jnp.dot(x, wq, preferred_element_type=jnp.float32)
    y = (acc + b_ref[0]).astype(jnp.bfloat16).astype(jnp.float32)
    o_ref[...] = _pack_halves(y)


def _gemm_body_alias(yin_ref, *refs, G):
    _gemm_body(*refs, G=G)


def _gemm_part(y_prev, xp, weight, weight_scale, bias, C, G, e_off):
    E, K, N = weight.shape
    NG = K // G
    ne = xp.shape[0] // C
    NS = 1
    NB = N // NS
    specs = [
        pl.BlockSpec((C, K // 2), lambda e, n: (e, 0)),
        pl.BlockSpec((1, K, NB), lambda e, n: (e + e_off, 0, n)),
        pl.BlockSpec((1, NG, NB), lambda e, n: (e + e_off, 0, n)),
        pl.BlockSpec((1, 1, NB), lambda e, n: (e + e_off, 0, n)),
    ]
    args = [xp, weight, weight_scale, bias.reshape(E, 1, N)]
    io_alias = {}
    body = functools.partial(_gemm_body, G=G)
    if y_prev is not None:
        specs = [pl.BlockSpec(memory_space=pltpu.MemorySpace.HBM)] + specs
        args = [y_prev] + args
        io_alias = {0: 0}
        body = functools.partial(_gemm_body_alias, G=G)
    return pl.pallas_call(
        body,
        grid=(ne, NS),
        in_specs=specs,
        out_specs=pl.BlockSpec((C, NB // 2), lambda e, n: (e + e_off, n)),
        out_shape=jax.ShapeDtypeStruct((E * C, N // 2), jnp.int32),
        input_output_aliases=io_alias,
        compiler_params=pltpu.CompilerParams(
            dimension_semantics=("arbitrary", "arbitrary")),
    )(*args)


def _combine_body(y_ref, g_ref, sh_ref, rsf_ref, o_ref):
    N = o_ref.shape[1]
    H = N // 2
    g = g_ref[...]
    rsf = rsf_ref[0, 0]
    s = jnp.maximum(g[:, 0:1] + g[:, 1:2], 1e-12)
    w = g * (rsf / s)
    p = y_ref[...]
    y0a, y0b = _unpack_halves(p[:, :H])
    y1a, y1b = _unpack_halves(p[:, H:])
    w0, w1 = w[:, 0:1], w[:, 1:2]
    sh = sh_ref[...]
    oa = w0 * y0a + w1 * y1a + sh[:, :H]
    ob = w0 * y0b + w1 * y1b + sh[:, H:]
    o_ref[...] = jnp.concatenate([oa, ob], axis=1).astype(jnp.bfloat16)


def _combine(yp2, gates, shared, rsf):
    T, N = yp2.shape
    B = 1024
    return pl.pallas_call(
        _combine_body,
        grid=(T // B,),
        in_specs=[
            pl.BlockSpec((B, N), lambda i: (i, 0)),
            pl.BlockSpec((B, 2), lambda i: (i, 0)),
            pl.BlockSpec((B, N), lambda i: (i, 0)),
            pl.BlockSpec((1, 1), lambda i: (0, 0)),
        ],
        out_specs=pl.BlockSpec((B, N), lambda i: (i, 0)),
        out_shape=jax.ShapeDtypeStruct((T, N), jnp.bfloat16),
        compiler_params=pltpu.CompilerParams(
            dimension_semantics=("arbitrary",)),
    )(yp2, gates, shared, rsf)


def kernel(input, weight, top_k_gates, token_indices, src_to_dst, token_count,
           shared_output, weight_scale, input_scale, bias,
           routed_scaling_factor):
    T, K = input.shape
    E, _, N = weight.shape
    total = token_indices.shape[0]
    C = total // E
    G = K // input_scale.shape[1]
    xp = _dequant_pack(input, input_scale, G)
    xdp = _sc_gather(xp, token_indices)
    yp = _gemm_part(None, xdp, weight, weight_scale, bias, C, G, 0)
    ysp = _sc_gather(yp, src_to_dst.reshape(-1))
    rsf = jnp.asarray(routed_scaling_factor, jnp.float32).reshape(1, 1)
    return _combine(ysp.reshape(T, N), top_k_gates, shared_output, rsf)


# speedup vs baseline: 1.3101x; 1.0858x over previous
import functools

import jax
import jax.numpy as jnp
from jax import lax
from jax.experimental import pallas as pl
from jax.experimental.pallas import tpu as pltpu
from jax.experimental.pallas import tpu_sc as plsc


def _pack_halves(y):
    bits = lax.bitcast_convert_type(y, jnp.int32)
    H = y.shape[1] // 2
    return lax.shift_right_logical(bits[:, :H], 16) | bits[:, H:]


def _unpack_halves(p):
    a = lax.bitcast_convert_type(p << 16, jnp.float32)
    b = lax.bitcast_convert_type(p & jnp.int32(-65536), jnp.float32)
    return a, b


def _sc_gather(table, idx, chunk=64, nbuf=3):
    B = idx.shape[0]
    D = table.shape[1]
    info = plsc.get_sparse_core_info()
    NC, NS = info.num_cores, info.num_subcores
    NW = NC * NS
    bpw = B // NW
    nch = bpw // chunk
    nbuf = min(nbuf, nch)
    idx3 = idx.reshape(NW, nch, chunk)

    scratch = [pltpu.VMEM((nch, chunk), jnp.int32)]
    scratch += [pltpu.VMEM((chunk, D), table.dtype) for _ in range(nbuf)]
    scratch += [pltpu.SemaphoreType.DMA for _ in range(2 * nbuf)]

    @functools.partial(
        pl.kernel,
        mesh=plsc.VectorSubcoreMesh(core_axis_name="c", subcore_axis_name="s"),
        out_type=jax.ShapeDtypeStruct((B, D), table.dtype),
        scratch_types=scratch,
    )
    def k(table_h, idx_h, out_h, idx_v, *rest):
        bufs = rest[:nbuf]
        gsem = rest[nbuf:2 * nbuf]
        osem = rest[2 * nbuf:3 * nbuf]
        wid = lax.axis_index("s") * NC + lax.axis_index("c")
        base = wid * bpw
        pltpu.sync_copy(idx_h.at[wid], idx_v)
        in_fl = {}
        out_fl = {}
        for b in range(nbuf):
            in_fl[b] = pltpu.async_copy(table_h.at[idx_v.at[b]], bufs[b], gsem[b])
        for i in range(nch):
            b = i % nbuf
            in_fl[b].wait()
            out_fl[b] = pltpu.async_copy(
                bufs[b], out_h.at[pl.ds(base + i * chunk, chunk)], osem[b])
            if i + nbuf < nch:
                out_fl[b].wait()
                in_fl[b] = pltpu.async_copy(
                    table_h.at[idx_v.at[i + nbuf]], bufs[b], gsem[b])
        for b in range(nbuf):
            out_fl[b].wait()

    return k(table, idx3)


def _group_onehot(NG, K, G):
    gidx = lax.broadcasted_iota(jnp.int32, (NG, K), 0)
    lidx = lax.broadcasted_iota(jnp.int32, (NG, K), 1)
    return (lidx // G == gidx).astype(jnp.float32)


def _deq_pack_body(x_ref, is_ref, o_ref, *, G):
    B, K = x_ref.shape
    NG = K // G
    srep = jnp.dot(is_ref[...], _group_onehot(NG, K, G),
                   preferred_element_type=jnp.float32)
    xq = (x_ref[...].astype(jnp.float32) * srep)
    o_ref[...] = _pack_halves(xq.astype(jnp.bfloat16).astype(jnp.float32))


def _dequant_pack(x, is_, G):
    T, K = x.shape
    NG = K // G
    B = 1024
    return pl.pallas_call(
        functools.partial(_deq_pack_body, G=G),
        grid=(T // B,),
        in_specs=[
            pl.BlockSpec((B, K), lambda i: (i, 0)),
            pl.BlockSpec((B, NG), lambda i: (i, 0)),
        ],
        out_specs=pl.BlockSpec((B, K // 2), lambda i: (i, 0)),
        out_shape=jax.ShapeDtypeStruct((T, K // 2), jnp.int32),
        compiler_params=pltpu.CompilerParams(
            dimension_semantics=("arbitrary",)),
    )(x, is_)


def _gemm_body(xp_ref, w_ref, ws_ref, b_ref, o_ref, *, G):
    EB, K, NB = w_ref.shape
    NG = K // G
    C = xp_ref.shape[0] // EB
    for i in range(EB):
        xa, xb = _unpack_halves(xp_ref[i * C:(i + 1) * C, :])
        x = jnp.concatenate([xa, xb], axis=1).astype(jnp.bfloat16)
        ws = ws_ref[i]
        wsrep = jnp.broadcast_to(ws[:, None, :], (NG, G, NB)).reshape(K, NB)
        wq = (w_ref[i].astype(jnp.float32) * wsrep).astype(jnp.bfloat16)
        acc = jnp.dot(x, wq, preferred_element_type=jnp.float32)
        y = (acc + b_ref[i]).astype(jnp.bfloat16).astype(jnp.float32)
        o_ref[i * C:(i + 1) * C, :] = _pack_halves(y)


def _gemm_body_alias(yin_ref, *refs, G):
    _gemm_body(*refs, G=G)


def _gemm_part(y_prev, xp, weight, weight_scale, bias, C, G, e_off):
    E, K, N = weight.shape
    NG = K // G
    ne = xp.shape[0] // C
    EB = 2
    NB = N
    specs = [
        pl.BlockSpec((EB * C, K // 2), lambda e, n: (e, 0)),
        pl.BlockSpec((EB, K, NB), lambda e, n: (e + e_off, 0, n)),
        pl.BlockSpec((EB, NG, NB), lambda e, n: (e + e_off, 0, n)),
        pl.BlockSpec((EB, 1, NB), lambda e, n: (e + e_off, 0, n)),
    ]
    args = [xp, weight, weight_scale, bias.reshape(E, 1, N)]
    io_alias = {}
    body = functools.partial(_gemm_body, G=G)
    if y_prev is not None:
        specs = [pl.BlockSpec(memory_space=pltpu.MemorySpace.HBM)] + specs
        args = [y_prev] + args
        io_alias = {0: 0}
        body = functools.partial(_gemm_body_alias, G=G)
    return pl.pallas_call(
        body,
        grid=(ne // EB, 1),
        in_specs=specs,
        out_specs=pl.BlockSpec((EB * C, NB // 2), lambda e, n: (e + e_off, n)),
        out_shape=jax.ShapeDtypeStruct((E * C, N // 2), jnp.int32),
        input_output_aliases=io_alias,
        compiler_params=pltpu.CompilerParams(
            dimension_semantics=("arbitrary", "arbitrary")),
    )(*args)


def _combine_body(y_ref, g_ref, sh_ref, rsf_ref, o_ref):
    N = o_ref.shape[1]
    H = N // 2
    g = g_ref[...]
    rsf = rsf_ref[0, 0]
    s = jnp.maximum(g[:, 0:1] + g[:, 1:2], 1e-12)
    w = g * (rsf / s)
    p = y_ref[...]
    y0a, y0b = _unpack_halves(p[:, :H])
    y1a, y1b = _unpack_halves(p[:, H:])
    w0, w1 = w[:, 0:1], w[:, 1:2]
    sh = sh_ref[...]
    oa = w0 * y0a + w1 * y1a + sh[:, :H]
    ob = w0 * y0b + w1 * y1b + sh[:, H:]
    o_ref[...] = jnp.concatenate([oa, ob], axis=1).astype(jnp.bfloat16)


def _combine(yp2, gates, shared, rsf):
    T, N = yp2.shape
    B = 1024
    return pl.pallas_call(
        _combine_body,
        grid=(T // B,),
        in_specs=[
            pl.BlockSpec((B, N), lambda i: (i, 0)),
            pl.BlockSpec((B, 2), lambda i: (i, 0)),
            pl.BlockSpec((B, N), lambda i: (i, 0)),
            pl.BlockSpec((1, 1), lambda i: (0, 0)),
        ],
        out_specs=pl.BlockSpec((B, N), lambda i: (i, 0)),
        out_shape=jax.ShapeDtypeStruct((T, N), jnp.bfloat16),
        compiler_params=pltpu.CompilerParams(
            dimension_semantics=("arbitrary",)),
    )(yp2, gates, shared, rsf)


def kernel(input, weight, top_k_gates, token_indices, src_to_dst, token_count,
           shared_output, weight_scale, input_scale, bias,
           routed_scaling_factor):
    T, K = input.shape
    E, _, N = weight.shape
    total = token_indices.shape[0]
    C = total // E
    G = K // input_scale.shape[1]
    xp = _dequant_pack(input, input_scale, G)
    xdp = _sc_gather(xp, token_indices)
    yp = _gemm_part(None, xdp, weight, weight_scale, bias, C, G, 0)
    ysp = _sc_gather(yp, src_to_dst.reshape(-1))
    rsf = jnp.asarray(routed_scaling_factor, jnp.float32).reshape(1, 1)
    return _combine(ysp.reshape(T, N), top_k_gates, shared_output, rsf)
